# Initial kernel scaffold; baseline (speedup 1.0000x reference)
import functools

import jax
import jax.numpy as jnp
from jax import lax
from jax.experimental import pallas as pl
from jax.experimental.pallas import tpu as pltpu
from jax.experimental.pallas import tpu_sc as plsc

N_NODES = 10000
N_EDGES = 320000
D_IN = 128
D_H = 192
N_LAYERS = 4
NUM_GRAPHS = 64
EPS = 1e-5

NC = 2
NS = 16
CHUNK = 128


@functools.lru_cache(maxsize=None)
def _make_edge_agg(d):
    per_w = N_EDGES // (NC * NS)
    nfull = per_w // CHUNK
    tail = per_w - nfull * CHUNK
    rows_t = N_NODES // NS
    mesh = plsc.VectorSubcoreMesh(core_axis_name="c", subcore_axis_name="s")

    @functools.partial(
        pl.kernel,
        out_type=jax.ShapeDtypeStruct((NC, N_NODES, d), jnp.float32),
        mesh=mesh,
        scratch_types=[
            pltpu.VMEM((CHUNK,), jnp.int32),
            pltpu.VMEM((CHUNK,), jnp.int32),
            pltpu.VMEM((CHUNK, d), jnp.float32),
            pltpu.VMEM((tail,), jnp.int32),
            pltpu.VMEM((tail,), jnp.int32),
            pltpu.VMEM((tail, d), jnp.float32),
            pltpu.VMEM_SHARED((N_NODES, d), jnp.float32),
            pltpu.SemaphoreType.DMA,
        ],
    )
    def edge_kernel(src_hbm, dst_hbm, h_hbm, zeros_hbm, out_hbm,
                    src_v, dst_v, rows_v, src_t, dst_t, rows_t_v, acc_sh, sem):
        c = lax.axis_index("c")
        s = lax.axis_index("s")
        pltpu.sync_copy(zeros_hbm, acc_sh.at[pl.ds(s * rows_t, rows_t)])
        plsc.subcore_barrier()

        base = (c * NS + s) * per_w

        @pl.loop(0, nfull)
        def _(g):
            b = base + g * CHUNK
            pltpu.sync_copy(src_hbm.at[pl.ds(b, CHUNK)], src_v)
            pltpu.sync_copy(dst_hbm.at[pl.ds(b, CHUNK)], dst_v)
            pltpu.async_copy(h_hbm.at[src_v], rows_v, sem).wait()
            pltpu.sync_copy(rows_v, acc_sh.at[dst_v], add=True)

        bt = base + nfull * CHUNK
        pltpu.sync_copy(src_hbm.at[pl.ds(bt, tail)], src_t)
        pltpu.sync_copy(dst_hbm.at[pl.ds(bt, tail)], dst_t)
        pltpu.async_copy(h_hbm.at[src_t], rows_t_v, sem).wait()
        pltpu.sync_copy(rows_t_v, acc_sh.at[dst_t], add=True)

        plsc.subcore_barrier()
        pltpu.sync_copy(acc_sh.at[pl.ds(s * rows_t, rows_t)],
                        out_hbm.at[c, pl.ds(s * rows_t, rows_t)])

    return edge_kernel


@functools.lru_cache(maxsize=None)
def _make_linear_stats(d):
    br = 1000
    nb = N_NODES // br

    def body(acc_ref, h_ref, wrel_ref, brel_ref, wroot_ref, h2_ref, stats_ref):
        b = pl.program_id(0)
        agg = acc_ref[0] + acc_ref[1]
        h2 = (jnp.dot(agg, wrel_ref[...], preferred_element_type=jnp.float32)
              + jnp.dot(h_ref[...], wroot_ref[...],
                        preferred_element_type=jnp.float32)
              + brel_ref[...])
        h2_ref[...] = h2

        @pl.when(b == 0)
        def _():
            stats_ref[...] = jnp.zeros_like(stats_ref)

        stats_ref[0:1, :] += jnp.sum(h2, axis=0, keepdims=True)
        stats_ref[1:2, :] += jnp.sum(h2 * h2, axis=0, keepdims=True)

    return pl.pallas_call(
        body,
        grid=(nb,),
        in_specs=[
            pl.BlockSpec((NC, br, d), lambda b: (0, b, 0)),
            pl.BlockSpec((br, d), lambda b: (b, 0)),
            pl.BlockSpec((d, D_H), lambda b: (0, 0)),
            pl.BlockSpec((1, D_H), lambda b: (0, 0)),
            pl.BlockSpec((d, D_H), lambda b: (0, 0)),
        ],
        out_specs=[
            pl.BlockSpec((br, D_H), lambda b: (b, 0)),
            pl.BlockSpec((2, D_H), lambda b: (0, 0)),
        ],
        out_shape=[
            jax.ShapeDtypeStruct((N_NODES, D_H), jnp.float32),
            jax.ShapeDtypeStruct((2, D_H), jnp.float32),
        ],
    )


def _make_bn_relu():
    br = 2000
    nb = N_NODES // br
    inv_n = 1.0 / N_NODES

    def body(h2_ref, stats_ref, gb_ref, out_ref):
        mean = stats_ref[0:1, :] * inv_n
        var = stats_ref[1:2, :] * inv_n - mean * mean
        scale = gb_ref[0:1, :] * lax.rsqrt(var + EPS)
        shift = gb_ref[1:2, :] - mean * scale
        out_ref[...] = jnp.maximum(h2_ref[...] * scale + shift, 0.0)

    return pl.pallas_call(
        body,
        grid=(nb,),
        in_specs=[
            pl.BlockSpec((br, D_H), lambda b: (b, 0)),
            pl.BlockSpec((2, D_H), lambda b: (0, 0)),
            pl.BlockSpec((2, D_H), lambda b: (0, 0)),
        ],
        out_specs=pl.BlockSpec((br, D_H), lambda b: (b, 0)),
        out_shape=jax.ShapeDtypeStruct((N_NODES, D_H), jnp.float32),
    )


_bn_relu = _make_bn_relu()


def _make_pool_head():
    def body(h_ref, bf_ref, w1_ref, b1_ref, w2_ref, b2_ref, wo_ref, bo_ref,
             out_ref):
        gids = lax.broadcasted_iota(jnp.float32, (NUM_GRAPHS, N_NODES), 0)
        eq = (bf_ref[...] == gids).astype(jnp.float32)
        sums = jnp.dot(eq, h_ref[...], preferred_element_type=jnp.float32)
        counts = jnp.sum(eq, axis=1, keepdims=True)
        g = sums / jnp.maximum(counts, 1.0)
        hh = jnp.maximum(
            jnp.dot(g, w1_ref[...], preferred_element_type=jnp.float32)
            + b1_ref[...], 0.0)
        hh = jnp.dot(hh, w2_ref[...],
                     preferred_element_type=jnp.float32) + b2_ref[...]
        out_ref[...] = jnp.dot(hh, wo_ref[...],
                               preferred_element_type=jnp.float32) + bo_ref[...]

    return pl.pallas_call(
        body,
        out_shape=jax.ShapeDtypeStruct((NUM_GRAPHS, 1), jnp.float32),
    )


_pool_head = _make_pool_head()


def kernel(x, edge_index, batch, params):
    src = edge_index[0].astype(jnp.int32)
    dst = edge_index[1].astype(jnp.int32)
    batch_f = batch.astype(jnp.float32).reshape(1, N_NODES)

    h = x
    d = D_IN
    for i in range(N_LAYERS):
        p = params[f'conv{i}']
        zeros = jnp.zeros((N_NODES // NS, d), dtype=jnp.float32)
        acc2 = _make_edge_agg(d)(src, dst, h, zeros)
        h2, stats = _make_linear_stats(d)(
            acc2, h, p['W_rel'], p['b_rel'].reshape(1, D_H), p['W_root'])
        gb = jnp.stack([p['gamma'], p['beta']])
        h = _bn_relu(h2, stats, gb)
        d = D_H

    hd = params['head']
    out = _pool_head(
        h, batch_f,
        hd['W1'], hd['b1'].reshape(1, D_H),
        hd['W2'], hd['b2'].reshape(1, D_H),
        params['out']['W'], params['out']['b'].reshape(1, 1))
    return out.reshape(-1)


# speedup vs baseline: 4.3415x; 4.3415x over previous
import functools

import jax
import jax.numpy as jnp
from jax import lax
from jax.experimental import pallas as pl
from jax.experimental.pallas import tpu as pltpu
from jax.experimental.pallas import tpu_sc as plsc

N_NODES = 10000
N_EDGES = 320000
D_IN = 128
D_H = 192
N_LAYERS = 4
NUM_GRAPHS = 64
EPS = 1e-5

NC = 2
NS = 16
CHUNK = 128
N_PAD = 10240


@functools.lru_cache(maxsize=None)
def _make_edge_agg(dh):
    per_w = N_EDGES // NS
    nfull = per_w // CHUNK
    tail = per_w - nfull * CHUNK
    rows_t = N_PAD // NS
    mesh = plsc.VectorSubcoreMesh(core_axis_name="c", subcore_axis_name="s")

    @functools.partial(
        pl.kernel,
        out_type=jax.ShapeDtypeStruct((NC, N_PAD, dh), jnp.float32),
        mesh=mesh,
        scratch_types=[
            pltpu.VMEM((CHUNK,), jnp.int32),
            pltpu.VMEM((CHUNK,), jnp.int32),
            pltpu.VMEM((CHUNK, dh), jnp.float32),
            pltpu.VMEM((tail,), jnp.int32),
            pltpu.VMEM((tail,), jnp.int32),
            pltpu.VMEM((tail, dh), jnp.float32),
            pltpu.VMEM_SHARED((N_PAD, dh), jnp.float32),
            pltpu.SemaphoreType.DMA,
        ],
        compiler_params=pltpu.CompilerParams(use_tc_tiling_on_sc=False),
    )
    def edge_kernel(src_hbm, dst_hbm, hlo_hbm, hhi_hbm, zeros_hbm, out_hbm,
                    src_v, dst_v, rows_v, src_t, dst_t, rows_t_v, acc_sh, sem):
        c = lax.axis_index("c")
        s = lax.axis_index("s")
        pltpu.sync_copy(zeros_hbm, acc_sh.at[pl.ds(s * rows_t, rows_t)])
        plsc.subcore_barrier()

        base = s * per_w

        @pl.loop(0, nfull)
        def _(g):
            b = base + g * CHUNK
            pltpu.sync_copy(src_hbm.at[pl.ds(b, CHUNK)], src_v)
            pltpu.sync_copy(dst_hbm.at[pl.ds(b, CHUNK)], dst_v)

            @pl.when(c == 0)
            def _():
                pltpu.async_copy(hlo_hbm.at[src_v], rows_v, sem).wait()

            @pl.when(c == 1)
            def _():
                pltpu.async_copy(hhi_hbm.at[src_v], rows_v, sem).wait()

            pltpu.sync_copy(rows_v, acc_sh.at[dst_v], add=True)

        bt = base + nfull * CHUNK
        pltpu.sync_copy(src_hbm.at[pl.ds(bt, tail)], src_t)
        pltpu.sync_copy(dst_hbm.at[pl.ds(bt, tail)], dst_t)

        @pl.when(c == 0)
        def _():
            pltpu.async_copy(hlo_hbm.at[src_t], rows_t_v, sem).wait()

        @pl.when(c == 1)
        def _():
            pltpu.async_copy(hhi_hbm.at[src_t], rows_t_v, sem).wait()

        pltpu.sync_copy(rows_t_v, acc_sh.at[dst_t], add=True)

        plsc.subcore_barrier()
        pltpu.sync_copy(acc_sh.at[pl.ds(s * rows_t, rows_t)],
                        out_hbm.at[c, pl.ds(s * rows_t, rows_t)])

    return edge_kernel


@functools.lru_cache(maxsize=None)
def _make_linear_stats(dh):
    br = 1000
    nb = N_NODES // br

    def body(acc_ref, hlo_ref, hhi_ref, wrel_ref, brel_ref, wroot_ref,
             h2_ref, stats_ref):
        b = pl.program_id(0)
        h2 = (jnp.dot(acc_ref[0], wrel_ref[0], preferred_element_type=jnp.float32)
              + jnp.dot(acc_ref[1], wrel_ref[1], preferred_element_type=jnp.float32)
              + jnp.dot(hlo_ref[...], wroot_ref[0],
                        preferred_element_type=jnp.float32)
              + jnp.dot(hhi_ref[...], wroot_ref[1],
                        preferred_element_type=jnp.float32)
              + brel_ref[...])
        h2_ref[...] = h2

        @pl.when(b == 0)
        def _():
            stats_ref[...] = jnp.zeros_like(stats_ref)

        stats_ref[0:1, :] += jnp.sum(h2, axis=0, keepdims=True)
        stats_ref[1:2, :] += jnp.sum(h2 * h2, axis=0, keepdims=True)

    return pl.pallas_call(
        body,
        grid=(nb,),
        in_specs=[
            pl.BlockSpec((NC, br, dh), lambda b: (0, b, 0)),
            pl.BlockSpec((br, dh), lambda b: (b, 0)),
            pl.BlockSpec((br, dh), lambda b: (b, 0)),
            pl.BlockSpec((NC, dh, D_H), lambda b: (0, 0, 0)),
            pl.BlockSpec((1, D_H), lambda b: (0, 0)),
            pl.BlockSpec((NC, dh, D_H), lambda b: (0, 0, 0)),
        ],
        out_specs=[
            pl.BlockSpec((br, D_H), lambda b: (b, 0)),
            pl.BlockSpec((2, D_H), lambda b: (0, 0)),
        ],
        out_shape=[
            jax.ShapeDtypeStruct((N_NODES, D_H), jnp.float32),
            jax.ShapeDtypeStruct((2, D_H), jnp.float32),
        ],
    )


def _make_bn_relu():
    br = 2000
    nb = N_NODES // br
    inv_n = 1.0 / N_NODES
    dh = D_H // 2

    def body(h2_ref, stats_ref, gb_ref, lo_ref, hi_ref):
        mean = stats_ref[0:1, :] * inv_n
        var = stats_ref[1:2, :] * inv_n - mean * mean
        scale = gb_ref[0:1, :] * lax.rsqrt(var + EPS)
        shift = gb_ref[1:2, :] - mean * scale
        res = jnp.maximum(h2_ref[...] * scale + shift, 0.0)
        lo_ref[...] = res[:, :dh]
        hi_ref[...] = res[:, dh:]

    return pl.pallas_call(
        body,
        grid=(nb,),
        in_specs=[
            pl.BlockSpec((br, D_H), lambda b: (b, 0)),
            pl.BlockSpec((2, D_H), lambda b: (0, 0)),
            pl.BlockSpec((2, D_H), lambda b: (0, 0)),
        ],
        out_specs=[
            pl.BlockSpec((br, dh), lambda b: (b, 0)),
            pl.BlockSpec((br, dh), lambda b: (b, 0)),
        ],
        out_shape=[
            jax.ShapeDtypeStruct((N_NODES, dh), jnp.float32),
            jax.ShapeDtypeStruct((N_NODES, dh), jnp.float32),
        ],
    )


_bn_relu = _make_bn_relu()


def _make_pool_head():
    dh = D_H // 2

    def body(hlo_ref, hhi_ref, bc_ref, w1_ref, b1_ref, w2_ref, b2_ref,
             wo_ref, bo_ref, out_ref):
        giota = lax.broadcasted_iota(
            jnp.int32, (N_NODES, NUM_GRAPHS), 1).astype(jnp.float32)
        eqt = (bc_ref[...] == giota).astype(jnp.float32)
        ones = jnp.zeros((N_NODES, 1), jnp.float32) + 1.0
        dn = (((0,), (0,)), ((), ()))
        counts = lax.dot_general(eqt, ones, dn,
                                 preferred_element_type=jnp.float32)
        inv = 1.0 / jnp.maximum(counts, 1.0)
        g0 = lax.dot_general(eqt, hlo_ref[...], dn,
                             preferred_element_type=jnp.float32) * inv
        g1 = lax.dot_general(eqt, hhi_ref[...], dn,
                             preferred_element_type=jnp.float32) * inv
        gp = jnp.concatenate(
            [g0, jnp.zeros((NUM_GRAPHS, 128 - dh), jnp.float32),
             g1, jnp.zeros((NUM_GRAPHS, 128 - dh), jnp.float32)], axis=1)
        hh = jnp.maximum(
            jnp.dot(gp, w1_ref[...], preferred_element_type=jnp.float32)
            + b1_ref[...], 0.0)
        hh = jnp.concatenate(
            [hh, jnp.zeros((NUM_GRAPHS, 256 - D_H), jnp.float32)], axis=1)
        hh = jnp.dot(hh, w2_ref[...],
                     preferred_element_type=jnp.float32) + b2_ref[...]
        hh = jnp.concatenate(
            [hh, jnp.zeros((NUM_GRAPHS, 256 - D_H), jnp.float32)], axis=1)
        out_ref[...] = jnp.dot(hh, wo_ref[...],
                               preferred_element_type=jnp.float32) + bo_ref[...]

    return pl.pallas_call(
        body,
        out_shape=jax.ShapeDtypeStruct((NUM_GRAPHS, 1), jnp.float32),
    )


_pool_head = _make_pool_head()


def _pad_rows(w, rows):
    return jnp.pad(w, ((0, rows - w.shape[0]), (0, 0)))


def _split2(w):
    d = w.shape[0]
    return jnp.stack([w[:d // 2], w[d // 2:]])


def kernel(x, edge_index, batch, params):
    src = edge_index[0].astype(jnp.int32)
    dst = edge_index[1].astype(jnp.int32)
    batch_c = batch.astype(jnp.float32).reshape(N_NODES, 1)

    hlo, hhi = x[:, :D_IN // 2], x[:, D_IN // 2:]
    d = D_IN
    for i in range(N_LAYERS):
        p = params[f'conv{i}']
        dh = d // 2
        zeros = jnp.zeros((N_PAD // NS, dh), dtype=jnp.float32)
        acc2 = _make_edge_agg(dh)(src, dst, hlo, hhi, zeros)
        h2, stats = _make_linear_stats(dh)(
            acc2, hlo, hhi, _split2(p['W_rel']), p['b_rel'].reshape(1, D_H),
            _split2(p['W_root']))
        gb = jnp.stack([p['gamma'], p['beta']])
        hlo, hhi = _bn_relu(h2, stats, gb)
        d = D_H

    hd = params['head']
    dh = D_H // 2
    w1p = jnp.concatenate(
        [hd['W1'][:dh], jnp.zeros((128 - dh, D_H), jnp.float32),
         hd['W1'][dh:], jnp.zeros((128 - dh, D_H), jnp.float32)], axis=0)
    out = _pool_head(
        hlo, hhi, batch_c,
        w1p, hd['b1'].reshape(1, D_H),
        _pad_rows(hd['W2'], 256), hd['b2'].reshape(1, D_H),
        _pad_rows(params['out']['W'], 256), params['out']['b'].reshape(1, 1))
    return out.reshape(-1)


# speedup vs baseline: 7.9340x; 1.8275x over previous
import functools

import jax
import jax.numpy as jnp
from jax import lax
from jax.experimental import pallas as pl
from jax.experimental.pallas import tpu as pltpu
from jax.experimental.pallas import tpu_sc as plsc

N_NODES = 10000
N_EDGES = 320000
D_IN = 128
D_H = 192
N_LAYERS = 4
NUM_GRAPHS = 64
EPS = 1e-5

NC = 2
NS = 16
CHUNK = 128
N_PAD = 10240
CPW = 157
E_PAD = NS * CPW * CHUNK


@functools.lru_cache(maxsize=None)
def _make_edge_agg(dh):
    rows_t = N_PAD // NS
    mesh = plsc.VectorSubcoreMesh(core_axis_name="c", subcore_axis_name="s")

    @functools.partial(
        pl.kernel,
        out_type=jax.ShapeDtypeStruct((NC, N_PAD, dh), jnp.float32),
        mesh=mesh,
        scratch_types=[
            pltpu.VMEM((CPW, CHUNK), jnp.int32),
            pltpu.VMEM((CPW, CHUNK), jnp.int32),
            pltpu.VMEM((CHUNK, dh), jnp.float32),
            pltpu.VMEM((CHUNK, dh), jnp.float32),
            pltpu.VMEM_SHARED((N_PAD, dh), jnp.float32),
            pltpu.SemaphoreType.DMA,
            pltpu.SemaphoreType.DMA,
            pltpu.SemaphoreType.DMA,
        ],
        compiler_params=pltpu.CompilerParams(use_tc_tiling_on_sc=False),
    )
    def edge_kernel(src_hbm, dst_hbm, hlo_hbm, hhi_hbm, zeros_hbm, out_hbm,
                    src_v, dst_v, rows_a, rows_b, acc_sh, gsem_a, gsem_b,
                    ssem):
        c = lax.axis_index("c")
        s = lax.axis_index("s")
        pltpu.sync_copy(zeros_hbm, acc_sh.at[pl.ds(s * rows_t, rows_t)])
        pltpu.sync_copy(src_hbm.at[pl.ds(s * CPW, CPW)], src_v)
        pltpu.sync_copy(dst_hbm.at[pl.ds(s * CPW, CPW)], dst_v)
        plsc.subcore_barrier()

        def gather(g, buf, sem):
            @pl.when(c == 0)
            def _():
                pltpu.async_copy(hlo_hbm.at[src_v.at[g]], buf, sem)

            @pl.when(c == 1)
            def _():
                pltpu.async_copy(hhi_hbm.at[src_v.at[g]], buf, sem)

        def wait_gather(buf, sem):
            pltpu.make_async_copy(hlo_hbm.at[src_v.at[0]], buf, sem).wait()

        def scatter(g, buf):
            pltpu.async_copy(buf, acc_sh.at[dst_v.at[g]], ssem, add=True)

        def wait_scatter(buf):
            pltpu.make_async_copy(buf, acc_sh.at[dst_v.at[0]], ssem).wait()

        gather(0, rows_a, gsem_a)

        @pl.loop(0, (CPW - 1) // 2)
        def _(t):
            a = 2 * t

            @pl.when(t > 0)
            def _():
                wait_scatter(rows_b)

            gather(a + 1, rows_b, gsem_b)
            wait_gather(rows_a, gsem_a)
            scatter(a, rows_a)
            wait_scatter(rows_a)
            gather(a + 2, rows_a, gsem_a)
            wait_gather(rows_b, gsem_b)
            scatter(a + 1, rows_b)

        wait_scatter(rows_b)
        wait_gather(rows_a, gsem_a)
        pltpu.sync_copy(rows_a, acc_sh.at[dst_v.at[CPW - 1]], add=True)

        plsc.subcore_barrier()
        pltpu.sync_copy(acc_sh.at[pl.ds(s * rows_t, rows_t)],
                        out_hbm.at[c, pl.ds(s * rows_t, rows_t)])

    return edge_kernel


@functools.lru_cache(maxsize=None)
def _make_linear_stats(dh):
    br = 1000
    nb = N_NODES // br

    def body(acc_ref, hlo_ref, hhi_ref, wrel_ref, brel_ref, wroot_ref,
             h2_ref, stats_ref):
        b = pl.program_id(0)
        h2 = (jnp.dot(acc_ref[0], wrel_ref[0], preferred_element_type=jnp.float32)
              + jnp.dot(acc_ref[1], wrel_ref[1], preferred_element_type=jnp.float32)
              + jnp.dot(hlo_ref[...], wroot_ref[0],
                        preferred_element_type=jnp.float32)
              + jnp.dot(hhi_ref[...], wroot_ref[1],
                        preferred_element_type=jnp.float32)
              + brel_ref[...])
        h2_ref[...] = h2

        @pl.when(b == 0)
        def _():
            stats_ref[...] = jnp.zeros_like(stats_ref)

        stats_ref[0:1, :] += jnp.sum(h2, axis=0, keepdims=True)
        stats_ref[1:2, :] += jnp.sum(h2 * h2, axis=0, keepdims=True)

    return pl.pallas_call(
        body,
        grid=(nb,),
        in_specs=[
            pl.BlockSpec((NC, br, dh), lambda b: (0, b, 0)),
            pl.BlockSpec((br, dh), lambda b: (b, 0)),
            pl.BlockSpec((br, dh), lambda b: (b, 0)),
            pl.BlockSpec((NC, dh, D_H), lambda b: (0, 0, 0)),
            pl.BlockSpec((1, D_H), lambda b: (0, 0)),
            pl.BlockSpec((NC, dh, D_H), lambda b: (0, 0, 0)),
        ],
        out_specs=[
            pl.BlockSpec((br, D_H), lambda b: (b, 0)),
            pl.BlockSpec((2, D_H), lambda b: (0, 0)),
        ],
        out_shape=[
            jax.ShapeDtypeStruct((N_NODES, D_H), jnp.float32),
            jax.ShapeDtypeStruct((2, D_H), jnp.float32),
        ],
    )


def _make_bn_relu():
    br = 2000
    nb = N_NODES // br
    inv_n = 1.0 / N_NODES
    dh = D_H // 2

    def body(h2_ref, stats_ref, gb_ref, lo_ref, hi_ref):
        mean = stats_ref[0:1, :] * inv_n
        var = stats_ref[1:2, :] * inv_n - mean * mean
        scale = gb_ref[0:1, :] / jnp.sqrt(var + EPS)
        shift = gb_ref[1:2, :] - mean * scale
        res = jnp.maximum(h2_ref[...] * scale + shift, 0.0)
        lo_ref[...] = res[:, :dh]
        hi_ref[...] = res[:, dh:]

    return pl.pallas_call(
        body,
        grid=(nb,),
        in_specs=[
            pl.BlockSpec((br, D_H), lambda b: (b, 0)),
            pl.BlockSpec((2, D_H), lambda b: (0, 0)),
            pl.BlockSpec((2, D_H), lambda b: (0, 0)),
        ],
        out_specs=[
            pl.BlockSpec((br, dh), lambda b: (b, 0)),
            pl.BlockSpec((br, dh), lambda b: (b, 0)),
        ],
        out_shape=[
            jax.ShapeDtypeStruct((N_NODES, dh), jnp.float32),
            jax.ShapeDtypeStruct((N_NODES, dh), jnp.float32),
        ],
    )


_bn_relu = _make_bn_relu()


def _make_pool_head():
    dh = D_H // 2

    def body(hlo_ref, hhi_ref, bc_ref, w1_ref, b1_ref, w2_ref, b2_ref,
             wo_ref, bo_ref, out_ref):
        giota = lax.broadcasted_iota(
            jnp.int32, (N_NODES, NUM_GRAPHS), 1).astype(jnp.float32)
        eqt = (bc_ref[...] == giota).astype(jnp.float32)
        ones = jnp.zeros((N_NODES, 1), jnp.float32) + 1.0
        dn = (((0,), (0,)), ((), ()))
        counts = lax.dot_general(eqt, ones, dn,
                                 preferred_element_type=jnp.float32)
        inv = 1.0 / jnp.maximum(counts, 1.0)
        g0 = lax.dot_general(eqt, hlo_ref[...], dn,
                             preferred_element_type=jnp.float32) * inv
        g1 = lax.dot_general(eqt, hhi_ref[...], dn,
                             preferred_element_type=jnp.float32) * inv
        gp = jnp.concatenate(
            [g0, jnp.zeros((NUM_GRAPHS, 128 - dh), jnp.float32),
             g1, jnp.zeros((NUM_GRAPHS, 128 - dh), jnp.float32)], axis=1)
        hh = jnp.maximum(
            jnp.dot(gp, w1_ref[...], preferred_element_type=jnp.float32)
            + b1_ref[...], 0.0)
        hh = jnp.concatenate(
            [hh, jnp.zeros((NUM_GRAPHS, 256 - D_H), jnp.float32)], axis=1)
        hh = jnp.dot(hh, w2_ref[...],
                     preferred_element_type=jnp.float32) + b2_ref[...]
        hh = jnp.concatenate(
            [hh, jnp.zeros((NUM_GRAPHS, 256 - D_H), jnp.float32)], axis=1)
        out_ref[...] = jnp.dot(hh, wo_ref[...],
                               preferred_element_type=jnp.float32) + bo_ref[...]

    return pl.pallas_call(
        body,
        out_shape=jax.ShapeDtypeStruct((NUM_GRAPHS, 1), jnp.float32),
    )


_pool_head = _make_pool_head()


def _pad_rows(w, rows):
    return jnp.pad(w, ((0, rows - w.shape[0]), (0, 0)))


def _split2(w):
    d = w.shape[0]
    return jnp.stack([w[:d // 2], w[d // 2:]])


def kernel(x, edge_index, batch, params):
    src = edge_index[0].astype(jnp.int32)
    dst = edge_index[1].astype(jnp.int32)
    batch_c = batch.astype(jnp.float32).reshape(N_NODES, 1)

    npad_e = E_PAD - N_EDGES
    src2d = jnp.concatenate(
        [src, jnp.zeros((npad_e,), jnp.int32)]).reshape(NS * CPW, CHUNK)
    dst2d = jnp.concatenate(
        [dst, jnp.full((npad_e,), N_NODES, jnp.int32)]).reshape(NS * CPW, CHUNK)

    hlo, hhi = x[:, :D_IN // 2], x[:, D_IN // 2:]
    d = D_IN
    for i in range(N_LAYERS):
        p = params[f'conv{i}']
        dh = d // 2
        zeros = jnp.zeros((N_PAD // NS, dh), dtype=jnp.float32)
        acc2 = _make_edge_agg(dh)(src2d, dst2d, hlo, hhi, zeros)
        h2, stats = _make_linear_stats(dh)(
            acc2, hlo, hhi, _split2(p['W_rel']), p['b_rel'].reshape(1, D_H),
            _split2(p['W_root']))
        gb = jnp.stack([p['gamma'], p['beta']])
        hlo, hhi = _bn_relu(h2, stats, gb)
        d = D_H

    hd = params['head']
    dh = D_H // 2
    w1p = jnp.concatenate(
        [hd['W1'][:dh], jnp.zeros((128 - dh, D_H), jnp.float32),
         hd['W1'][dh:], jnp.zeros((128 - dh, D_H), jnp.float32)], axis=0)
    out = _pool_head(
        hlo, hhi, batch_c,
        w1p, hd['b1'].reshape(1, D_H),
        _pad_rows(hd['W2'], 256), hd['b2'].reshape(1, D_H),
        _pad_rows(params['out']['W'], 256), params['out']['b'].reshape(1, 1))
    return out.reshape(-1)


# speedup vs baseline: 8.5617x; 1.0791x over previous
import functools

import jax
import jax.numpy as jnp
from jax import lax
from jax.experimental import pallas as pl
from jax.experimental.pallas import tpu as pltpu
from jax.experimental.pallas import tpu_sc as plsc

N_NODES = 10000
N_EDGES = 320000
D_IN = 128
D_H = 192
N_LAYERS = 4
NUM_GRAPHS = 64
EPS = 1e-5

NC = 2
NS = 16
CHUNK = 128
N_PAD = 10240
CPW = 157
E_PAD = NS * CPW * CHUNK


@functools.lru_cache(maxsize=None)
def _make_edge_agg(dh):
    rows_t = N_PAD // NS
    mesh = plsc.VectorSubcoreMesh(core_axis_name="c", subcore_axis_name="s")

    @functools.partial(
        pl.kernel,
        out_type=jax.ShapeDtypeStruct((NC, N_PAD, dh), jnp.float32),
        mesh=mesh,
        scratch_types=[
            pltpu.VMEM((CPW, CHUNK), jnp.int32),
            pltpu.VMEM((CHUNK,), jnp.int32),
            pltpu.VMEM((CHUNK,), jnp.int32),
            pltpu.VMEM((CHUNK,), jnp.int32),
            pltpu.VMEM((CHUNK, dh), jnp.float32),
            pltpu.VMEM((CHUNK, dh), jnp.float32),
            pltpu.VMEM((CHUNK, dh), jnp.float32),
            pltpu.VMEM_SHARED((N_PAD, dh), jnp.float32),
            pltpu.SemaphoreType.DMA,
            pltpu.SemaphoreType.DMA,
            pltpu.SemaphoreType.DMA,
            pltpu.SemaphoreType.DMA,
            pltpu.SemaphoreType.DMA,
            pltpu.SemaphoreType.DMA,
            pltpu.SemaphoreType.DMA,
            pltpu.SemaphoreType.DMA,
            pltpu.SemaphoreType.DMA,
        ],
        compiler_params=pltpu.CompilerParams(use_tc_tiling_on_sc=False),
    )
    def edge_kernel(src_hbm, dst_hbm, hlo_hbm, hhi_hbm, zeros_hbm, out_hbm,
                    src_v, db0, db1, db2, buf0, buf1, buf2, acc_sh,
                    gs0, gs1, gs2, ss0, ss1, ss2, ds0, ds1, ds2):
        bufs = (buf0, buf1, buf2)
        dbufs = (db0, db1, db2)
        gsems = (gs0, gs1, gs2)
        ssems = (ss0, ss1, ss2)
        dsems = (ds0, ds1, ds2)
        c = lax.axis_index("c")
        s = lax.axis_index("s")
        pltpu.sync_copy(zeros_hbm, acc_sh.at[pl.ds(s * rows_t, rows_t)])
        pltpu.sync_copy(src_hbm.at[pl.ds(s * CPW, CPW)], src_v)
        plsc.subcore_barrier()

        def gather(g, b):
            pltpu.async_copy(dst_hbm.at[s * CPW + g], dbufs[b], dsems[b])

            @pl.when(c == 0)
            def _():
                pltpu.async_copy(hlo_hbm.at[src_v.at[g]], bufs[b], gsems[b])

            @pl.when(c == 1)
            def _():
                pltpu.async_copy(hhi_hbm.at[src_v.at[g]], bufs[b], gsems[b])

        def wait_gather(b):
            pltpu.make_async_copy(
                dst_hbm.at[0], dbufs[b], dsems[b]).wait()
            pltpu.make_async_copy(
                hlo_hbm.at[src_v.at[0]], bufs[b], gsems[b]).wait()

        def scatter(g, b):
            pltpu.async_copy(bufs[b], acc_sh.at[dbufs[b]], ssems[b],
                             add=True)

        def wait_scatter(b):
            pltpu.make_async_copy(
                bufs[b], acc_sh.at[dbufs[b]], ssems[b]).wait()

        gather(0, 0)
        gather(1, 1)
        nt = (CPW - 1) // 3

        @pl.loop(0, nt)
        def _(t):
            for b in range(3):
                wait_gather(b)
                g = 3 * t + b
                scatter(g, b)
                prev = (b - 1) % 3
                if b == 0:
                    @pl.when(t > 0)
                    def _():
                        wait_scatter(prev)
                        gather(g + 2, prev)

                    @pl.when(t == 0)
                    def _():
                        gather(g + 2, prev)
                elif b == 1:
                    wait_scatter(prev)
                    gather(g + 2, prev)
                else:
                    wait_scatter(prev)

                    @pl.when(t < nt - 1)
                    def _():
                        gather(g + 2, prev)

        wait_scatter(2)
        wait_gather(0)
        pltpu.sync_copy(bufs[0], acc_sh.at[dbufs[0]], add=True)

        plsc.subcore_barrier()
        pltpu.sync_copy(acc_sh.at[pl.ds(s * rows_t, rows_t)],
                        out_hbm.at[c, pl.ds(s * rows_t, rows_t)])

    return edge_kernel


@functools.lru_cache(maxsize=None)
def _make_linear_stats(dh):
    br = 1000
    nb = N_NODES // br

    def body(acc_ref, hlo_ref, hhi_ref, wrel_ref, brel_ref, wroot_ref,
             h2_ref, stats_ref):
        b = pl.program_id(0)
        h2 = (jnp.dot(acc_ref[0], wrel_ref[0], preferred_element_type=jnp.float32)
              + jnp.dot(acc_ref[1], wrel_ref[1], preferred_element_type=jnp.float32)
              + jnp.dot(hlo_ref[...], wroot_ref[0],
                        preferred_element_type=jnp.float32)
              + jnp.dot(hhi_ref[...], wroot_ref[1],
                        preferred_element_type=jnp.float32)
              + brel_ref[...])
        h2_ref[...] = h2

        @pl.when(b == 0)
        def _():
            stats_ref[...] = jnp.zeros_like(stats_ref)

        stats_ref[0:1, :] += jnp.sum(h2, axis=0, keepdims=True)
        stats_ref[1:2, :] += jnp.sum(h2 * h2, axis=0, keepdims=True)

    return pl.pallas_call(
        body,
        grid=(nb,),
        in_specs=[
            pl.BlockSpec((NC, br, dh), lambda b: (0, b, 0)),
            pl.BlockSpec((br, dh), lambda b: (b, 0)),
            pl.BlockSpec((br, dh), lambda b: (b, 0)),
            pl.BlockSpec((NC, dh, D_H), lambda b: (0, 0, 0)),
            pl.BlockSpec((1, D_H), lambda b: (0, 0)),
            pl.BlockSpec((NC, dh, D_H), lambda b: (0, 0, 0)),
        ],
        out_specs=[
            pl.BlockSpec((br, D_H), lambda b: (b, 0)),
            pl.BlockSpec((2, D_H), lambda b: (0, 0)),
        ],
        out_shape=[
            jax.ShapeDtypeStruct((N_NODES, D_H), jnp.float32),
            jax.ShapeDtypeStruct((2, D_H), jnp.float32),
        ],
    )


def _make_bn_relu():
    br = 2000
    nb = N_NODES // br
    inv_n = 1.0 / N_NODES
    dh = D_H // 2

    def body(h2_ref, stats_ref, gb_ref, lo_ref, hi_ref):
        mean = stats_ref[0:1, :] * inv_n
        var = stats_ref[1:2, :] * inv_n - mean * mean
        scale = gb_ref[0:1, :] / jnp.sqrt(var + EPS)
        shift = gb_ref[1:2, :] - mean * scale
        res = jnp.maximum(h2_ref[...] * scale + shift, 0.0)
        lo_ref[...] = res[:, :dh]
        hi_ref[...] = res[:, dh:]

    return pl.pallas_call(
        body,
        grid=(nb,),
        in_specs=[
            pl.BlockSpec((br, D_H), lambda b: (b, 0)),
            pl.BlockSpec((2, D_H), lambda b: (0, 0)),
            pl.BlockSpec((2, D_H), lambda b: (0, 0)),
        ],
        out_specs=[
            pl.BlockSpec((br, dh), lambda b: (b, 0)),
            pl.BlockSpec((br, dh), lambda b: (b, 0)),
        ],
        out_shape=[
            jax.ShapeDtypeStruct((N_NODES, dh), jnp.float32),
            jax.ShapeDtypeStruct((N_NODES, dh), jnp.float32),
        ],
    )


_bn_relu = _make_bn_relu()


def _make_pool_head():
    dh = D_H // 2

    def body(hlo_ref, hhi_ref, bc_ref, w1_ref, b1_ref, w2_ref, b2_ref,
             wo_ref, bo_ref, out_ref):
        giota = lax.broadcasted_iota(
            jnp.int32, (N_NODES, NUM_GRAPHS), 1).astype(jnp.float32)
        eqt = (bc_ref[...] == giota).astype(jnp.float32)
        ones = jnp.zeros((N_NODES, 1), jnp.float32) + 1.0
        dn = (((0,), (0,)), ((), ()))
        counts = lax.dot_general(eqt, ones, dn,
                                 preferred_element_type=jnp.float32)
        inv = 1.0 / jnp.maximum(counts, 1.0)
        g0 = lax.dot_general(eqt, hlo_ref[...], dn,
                             preferred_element_type=jnp.float32) * inv
        g1 = lax.dot_general(eqt, hhi_ref[...], dn,
                             preferred_element_type=jnp.float32) * inv
        gp = jnp.concatenate(
            [g0, jnp.zeros((NUM_GRAPHS, 128 - dh), jnp.float32),
             g1, jnp.zeros((NUM_GRAPHS, 128 - dh), jnp.float32)], axis=1)
        hh = jnp.maximum(
            jnp.dot(gp, w1_ref[...], preferred_element_type=jnp.float32)
            + b1_ref[...], 0.0)
        hh = jnp.concatenate(
            [hh, jnp.zeros((NUM_GRAPHS, 256 - D_H), jnp.float32)], axis=1)
        hh = jnp.dot(hh, w2_ref[...],
                     preferred_element_type=jnp.float32) + b2_ref[...]
        hh = jnp.concatenate(
            [hh, jnp.zeros((NUM_GRAPHS, 256 - D_H), jnp.float32)], axis=1)
        out_ref[...] = jnp.dot(hh, wo_ref[...],
                               preferred_element_type=jnp.float32) + bo_ref[...]

    return pl.pallas_call(
        body,
        out_shape=jax.ShapeDtypeStruct((NUM_GRAPHS, 1), jnp.float32),
    )


_pool_head = _make_pool_head()


def _pad_rows(w, rows):
    return jnp.pad(w, ((0, rows - w.shape[0]), (0, 0)))


def _split2(w):
    d = w.shape[0]
    return jnp.stack([w[:d // 2], w[d // 2:]])


def kernel(x, edge_index, batch, params):
    src = edge_index[0].astype(jnp.int32)
    dst = edge_index[1].astype(jnp.int32)
    batch_c = batch.astype(jnp.float32).reshape(N_NODES, 1)

    npad_e = E_PAD - N_EDGES
    src2d = jnp.concatenate(
        [src, jnp.zeros((npad_e,), jnp.int32)]).reshape(NS * CPW, CHUNK)
    dst2d = jnp.concatenate(
        [dst, jnp.full((npad_e,), N_NODES, jnp.int32)]).reshape(NS * CPW, CHUNK)

    hlo, hhi = x[:, :D_IN // 2], x[:, D_IN // 2:]
    d = D_IN
    for i in range(N_LAYERS):
        p = params[f'conv{i}']
        dh = d // 2
        zeros = jnp.zeros((N_PAD // NS, dh), dtype=jnp.float32)
        acc2 = _make_edge_agg(dh)(src2d, dst2d, hlo, hhi, zeros)
        h2, stats = _make_linear_stats(dh)(
            acc2, hlo, hhi, _split2(p['W_rel']), p['b_rel'].reshape(1, D_H),
            _split2(p['W_root']))
        gb = jnp.stack([p['gamma'], p['beta']])
        hlo, hhi = _bn_relu(h2, stats, gb)
        d = D_H

    hd = params['head']
    dh = D_H // 2
    w1p = jnp.concatenate(
        [hd['W1'][:dh], jnp.zeros((128 - dh, D_H), jnp.float32),
         hd['W1'][dh:], jnp.zeros((128 - dh, D_H), jnp.float32)], axis=0)
    out = _pool_head(
        hlo, hhi, batch_c,
        w1p, hd['b1'].reshape(1, D_H),
        _pad_rows(hd['W2'], 256), hd['b2'].reshape(1, D_H),
        _pad_rows(params['out']['W'], 256), params['out']['b'].reshape(1, 1))
    return out.reshape(-1)
